# baseline (speedup 1.0000x reference)
import functools

import jax
import jax.numpy as jnp
from jax import lax
from jax.experimental import pallas as pl
from jax.experimental.pallas import tpu as pltpu, tpu_sc as plsc

NC = 2
NS = 16
NW = NC * NS
C = 128
PACK_SHIFT = 14

_SC_PARAMS = pltpu.CompilerParams(
    needs_layout_passes=False, use_tc_tiling_on_sc=False)


def _pad_chunks(arr, fill, nbuf):
    e = arr.shape[0]
    ept = e // NW
    k = -(-(-(-ept // C)) // nbuf) * nbuf
    kt = k + nbuf
    pad = kt * C - ept
    out = jnp.concatenate(
        [arr.reshape(NW, ept),
         jnp.full((NW, pad), fill, jnp.int32)], axis=1)
    return out.reshape(NW, kt, C), k


def _make_hist_kernel(n_pad, k):
    mesh = plsc.VectorSubcoreMesh(core_axis_name="c", subcore_axis_name="s")

    @functools.partial(
        pl.kernel,
        out_type=jax.ShapeDtypeStruct((NW, n_pad), jnp.float32),
        mesh=mesh,
        scratch_types=[
            pltpu.VMEM((n_pad,), jnp.float32),
            pltpu.VMEM((k, C), jnp.int32),
        ],
        compiler_params=_SC_PARAMS,
    )
    def hist_kernel(dst_hbm, out_hbm, hist_v, idx_v):
        cid = lax.axis_index("c")
        sid = lax.axis_index("s")
        wid = sid * NC + cid

        pltpu.sync_copy(dst_hbm.at[wid, pl.ds(0, k)], idx_v)

        def zero_body(i, _):
            hist_v[pl.ds(i * 16, 16)] = jnp.zeros((16,), jnp.float32)
            return 0

        lax.fori_loop(0, n_pad // 16, zero_body, 0)

        ones = jnp.ones((16,), jnp.float32)

        def chunk_body(kk, _):
            for j in range(C // 16):
                iv = idx_v[kk, pl.ds(j * 16, 16)]
                plsc.addupdate_scatter(hist_v, [iv], ones)
            return 0

        lax.fori_loop(0, k, chunk_body, 0)
        pltpu.sync_copy(hist_v, out_hbm.at[wid])

    return hist_kernel


def _make_prop_kernel(n_pad, k, feat, nbuf, packed):
    mesh = plsc.VectorSubcoreMesh(core_axis_name="c", subcore_axis_name="s")
    rz = n_pad // NS
    H = max(nbuf // 2, 1)

    if packed:
        idx_scratch = [
            pltpu.VMEM((k + nbuf, C), jnp.int32),
            [pltpu.VMEM((C,), jnp.int32) for _ in range(nbuf)],
            [pltpu.VMEM((C,), jnp.int32) for _ in range(nbuf)],
        ]
    else:
        idx_scratch = [
            pltpu.VMEM((k + nbuf, C), jnp.int32),
            pltpu.VMEM((k + nbuf, C), jnp.int32),
        ]

    @functools.partial(
        pl.kernel,
        out_type=jax.ShapeDtypeStruct((NC, n_pad, feat), jnp.float32),
        mesh=mesh,
        scratch_types=idx_scratch + [
            [pltpu.VMEM((C, feat), jnp.float32) for _ in range(nbuf)],
            [pltpu.SemaphoreType.DMA for _ in range(nbuf)],
            [pltpu.SemaphoreType.DMA for _ in range(nbuf)],
            pltpu.VMEM_SHARED((n_pad, feat), jnp.float32),
        ],
        compiler_params=_SC_PARAMS,
    )
    def prop_kernel(idx_hbm, table_hbm, zeros_hbm, out_hbm, *refs):
        if packed:
            idx_p, src_ring, dst_ring, rows, sem_g, sem_s, acc_sh = refs
        else:
            idx_s, idx_d, rows, sem_g, sem_s, acc_sh = refs
        cid = lax.axis_index("c")
        sid = lax.axis_index("s")
        wid = sid * NC + cid

        if packed:
            pltpu.sync_copy(idx_hbm.at[wid], idx_p)

            def unpack(kk, r):
                for j in range(C // 16):
                    v = idx_p[kk, pl.ds(j * 16, 16)]
                    src_ring[r][pl.ds(j * 16, 16)] = jnp.bitwise_and(
                        v, (1 << PACK_SHIFT) - 1)
                    dst_ring[r][pl.ds(j * 16, 16)] = lax.shift_right_logical(
                        v, PACK_SHIFT)

            def src_at(kk, r):
                del kk
                return src_ring[r]

            def dst_at(kk, r):
                del kk
                return dst_ring[r]
        else:
            pltpu.sync_copy(idx_hbm.at[0, wid], idx_s)
            pltpu.sync_copy(idx_hbm.at[1, wid], idx_d)

            def unpack(kk, r):
                del kk, r

            def src_at(kk, r):
                del r
                return idx_s.at[kk]

            def dst_at(kk, r):
                del r
                return idx_d.at[kk]

        for b in range(H):
            unpack(b, b)
            pltpu.async_copy(table_hbm.at[src_at(b, b)], rows[b], sem_g[b])
        pltpu.sync_copy(zeros_hbm.at[pl.ds(sid * rz, rz)],
                        acc_sh.at[pl.ds(sid * rz, rz)])
        plsc.subcore_barrier()

        def stage(kk, b, first=False):
            bn = (b + H) % nbuf
            if not first:
                pltpu.make_async_copy(
                    rows[bn], acc_sh.at[dst_at(0, bn)], sem_s[bn]).wait()
            unpack(kk + H, bn)
            pltpu.make_async_copy(
                table_hbm.at[src_at(kk, b)], rows[b], sem_g[b]).wait()
            pltpu.async_copy(rows[b], acc_sh.at[dst_at(kk, b)], sem_s[b],
                             add=True)
            pltpu.async_copy(
                table_hbm.at[src_at(kk + H, bn)], rows[bn], sem_g[bn])

        for b in range(nbuf):
            stage(b, b, first=b < H)

        def group_body(g, _):
            for b in range(nbuf):
                stage(g * nbuf + b, b)
            return 0

        lax.fori_loop(1, k // nbuf, group_body, 0)
        for j in range(H):
            bs = (k - H + j) % nbuf
            pltpu.make_async_copy(
                rows[bs], acc_sh.at[dst_at(0, bs)], sem_s[bs]).wait()
            bg = (k + j) % nbuf
            pltpu.make_async_copy(
                table_hbm.at[src_at(k + j, bg)], rows[bg], sem_g[bg]).wait()
        plsc.subcore_barrier()
        pltpu.sync_copy(acc_sh.at[pl.ds(sid * rz, rz)],
                        out_hbm.at[cid, pl.ds(sid * rz, rz)])

    return prop_kernel


def _xs_body(hist_ref, x_ref, o_ref):
    deg = jnp.sum(hist_ref[...], axis=1) + 1.0
    dinv = lax.rsqrt(deg)
    o_ref[...] = x_ref[...] * dinv[:, None]


def _mid_body(hist_ref, part_ref, xs_ref, w1_ref, b1_ref, w2_ref, o_ref):
    deg = jnp.sum(hist_ref[...], axis=1) + 1.0
    dinv = lax.rsqrt(deg)
    p1 = dinv[:, None] * (part_ref[0] + part_ref[1] + xs_ref[...])
    h = jnp.maximum(
        jnp.dot(p1, w1_ref[...], preferred_element_type=jnp.float32)
        + b1_ref[0:1, :], 0.0)
    g = jnp.dot(h, w2_ref[...], preferred_element_type=jnp.float32)
    o_ref[...] = g * dinv[:, None]


def _final_body(hist_ref, part_ref, gs_ref, b2_ref, o_ref):
    deg = jnp.sum(hist_ref[...], axis=1) + 1.0
    dinv = lax.rsqrt(deg)
    o_ref[...] = dinv[:, None] * (part_ref[0] + part_ref[1] + gs_ref[...]) \
        + b2_ref[0:1, :]


def kernel(x, edge_index, W1, b1, W2, b2):
    n, d_in = x.shape
    d_hid = W1.shape[1]
    n_cls = W2.shape[1]
    src = edge_index[0]
    dst = edge_index[1]

    n_pad = -(-(n + 1) // 128) * 128

    nb1, nb2 = 2, 8
    srcp, kp = _pad_chunks(src, 0, nb1)
    dstp, _ = _pad_chunks(dst, n, nb1)
    packed = srcp + (dstp << PACK_SHIFT)
    src8, k8 = _pad_chunks(src, 0, nb2)
    dst8, _ = _pad_chunks(dst, n, nb2)
    sd8 = jnp.stack([src8, dst8])

    hist = _make_hist_kernel(n_pad, k8)(dst8)
    hist_n = hist[:, :n].T

    br = 1000
    grid = (n // br,)
    hist_spec = pl.BlockSpec((br, NW), lambda i: (i, 0))
    row_spec = lambda f: pl.BlockSpec((br, f), lambda i: (i, 0))
    part_spec = lambda f: pl.BlockSpec((NC, br, f), lambda i: (0, i, 0))

    xs = pl.pallas_call(
        _xs_body,
        grid=grid,
        in_specs=[hist_spec, row_spec(d_in)],
        out_specs=row_spec(d_in),
        out_shape=jax.ShapeDtypeStruct((n, d_in), jnp.float32),
    )(hist_n, x)

    zeros_in = jnp.zeros((n_pad, d_in), jnp.float32)
    part1 = _make_prop_kernel(n_pad, kp, d_in, nb1, True)(
        packed, xs, zeros_in)

    b1r = jnp.broadcast_to(b1[None, :], (8, d_hid))
    full = lambda shape: pl.BlockSpec(shape, lambda i: tuple(0 for _ in shape))
    gs = pl.pallas_call(
        _mid_body,
        grid=grid,
        in_specs=[hist_spec, part_spec(d_in), row_spec(d_in),
                  full((d_in, d_hid)), full((8, d_hid)), full((d_hid, n_cls))],
        out_specs=row_spec(n_cls),
        out_shape=jax.ShapeDtypeStruct((n, n_cls), jnp.float32),
    )(hist_n, part1, xs, W1, b1r, W2)

    zeros_cls = jnp.zeros((n_pad, n_cls), jnp.float32)
    part2 = _make_prop_kernel(n_pad, k8, n_cls, nb2, False)(
        sd8, gs, zeros_cls)

    b2r = jnp.broadcast_to(b2[None, :], (8, n_cls))
    out = pl.pallas_call(
        _final_body,
        grid=grid,
        in_specs=[hist_spec, part_spec(n_cls), row_spec(n_cls), full((8, n_cls))],
        out_specs=row_spec(n_cls),
        out_shape=jax.ShapeDtypeStruct((n, n_cls), jnp.float32),
    )(hist_n, part2, gs, b2r)
    return out


# speedup vs baseline: 1.4192x; 1.4192x over previous
import functools

import jax
import jax.numpy as jnp
from jax import lax
from jax.experimental import pallas as pl
from jax.experimental.pallas import tpu as pltpu, tpu_sc as plsc

NC = 2
NS = 16
NW = NC * NS
C = 128
NBUF = 4

_SC_PARAMS = pltpu.CompilerParams(
    needs_layout_passes=False, use_tc_tiling_on_sc=False)


def _pad_chunks(arr, workers, fill):
    e = arr.shape[0]
    ept = e // workers
    k = -(-(-(-ept // C)) // NBUF) * NBUF
    kt = k + NBUF
    pad = kt * C - ept
    out = jnp.concatenate(
        [arr.reshape(workers, ept),
         jnp.full((workers, pad), fill, jnp.int32)], axis=1)
    return out.reshape(workers, kt, C), k


def _make_hist_kernel(n_pad, k):
    mesh = plsc.VectorSubcoreMesh(core_axis_name="c", subcore_axis_name="s")

    @functools.partial(
        pl.kernel,
        out_type=jax.ShapeDtypeStruct((NW, n_pad), jnp.float32),
        mesh=mesh,
        scratch_types=[
            pltpu.VMEM((n_pad,), jnp.float32),
            pltpu.VMEM((k, C), jnp.int32),
        ],
        compiler_params=_SC_PARAMS,
    )
    def hist_kernel(dst_hbm, out_hbm, hist_v, idx_v):
        cid = lax.axis_index("c")
        sid = lax.axis_index("s")
        wid = sid * NC + cid

        pltpu.sync_copy(dst_hbm.at[wid, pl.ds(0, k)], idx_v)

        def zero_body(i, _):
            hist_v[pl.ds(i * 16, 16)] = jnp.zeros((16,), jnp.float32)
            return 0

        lax.fori_loop(0, n_pad // 16, zero_body, 0)

        ones = jnp.ones((16,), jnp.float32)

        def chunk_body(kk, _):
            for j in range(C // 16):
                iv = idx_v[kk, pl.ds(j * 16, 16)]
                plsc.addupdate_scatter(hist_v, [iv], ones)
            return 0

        lax.fori_loop(0, k, chunk_body, 0)
        pltpu.sync_copy(hist_v, out_hbm.at[wid])

    return hist_kernel


def _make_prop_kernel(n_pad, k, feat, feature_split):
    mesh = plsc.VectorSubcoreMesh(core_axis_name="c", subcore_axis_name="s")
    rz = n_pad // NS
    H = NBUF // 2

    @functools.partial(
        pl.kernel,
        out_type=jax.ShapeDtypeStruct((NC, n_pad, feat), jnp.float32),
        mesh=mesh,
        scratch_types=[
            pltpu.VMEM((k + NBUF, C), jnp.int32),
            pltpu.VMEM((k + NBUF, C), jnp.int32),
            [pltpu.VMEM((C, feat), jnp.float32) for _ in range(NBUF)],
            [pltpu.SemaphoreType.DMA for _ in range(NBUF)],
            [pltpu.SemaphoreType.DMA for _ in range(NBUF)],
            pltpu.VMEM_SHARED((n_pad, feat), jnp.float32),
        ],
        compiler_params=_SC_PARAMS,
    )
    def prop_kernel(src_hbm, dst_hbm, table_hbm, zeros_hbm, out_hbm,
                    idx_s, idx_d, rows, sem_g, sem_s, acc_sh):
        cid = lax.axis_index("c")
        sid = lax.axis_index("s")

        if feature_split:
            pltpu.sync_copy(src_hbm.at[cid, sid], idx_s)
            pltpu.sync_copy(dst_hbm.at[sid], idx_d)
        else:
            wid = sid * NC + cid
            pltpu.sync_copy(src_hbm.at[wid], idx_s)
            pltpu.sync_copy(dst_hbm.at[wid], idx_d)
        for b in range(H):
            pltpu.async_copy(table_hbm.at[idx_s.at[b]], rows[b], sem_g[b])
        pltpu.sync_copy(zeros_hbm.at[pl.ds(sid * rz, rz)],
                        acc_sh.at[pl.ds(sid * rz, rz)])
        plsc.subcore_barrier()

        def stage(kk, b, skip_scatter_wait=False):
            bn = (b + H) % NBUF
            pltpu.make_async_copy(
                table_hbm.at[idx_s.at[kk]], rows[b], sem_g[b]).wait()
            pltpu.async_copy(rows[b], acc_sh.at[idx_d.at[kk]], sem_s[b],
                             add=True)
            if not skip_scatter_wait:
                pltpu.make_async_copy(
                    rows[bn], acc_sh.at[idx_d.at[0]], sem_s[bn]).wait()
            pltpu.async_copy(
                table_hbm.at[idx_s.at[kk + H]], rows[bn], sem_g[bn])

        for b in range(NBUF):
            stage(b, b, skip_scatter_wait=b < H)

        def group_body(g, _):
            for b in range(NBUF):
                stage(g * NBUF + b, b)
            return 0

        lax.fori_loop(1, k // NBUF, group_body, 0)
        for j in range(H):
            bs = (k - H + j) % NBUF
            pltpu.make_async_copy(
                rows[bs], acc_sh.at[idx_d.at[0]], sem_s[bs]).wait()
            bg = (k + j) % NBUF
            pltpu.make_async_copy(
                table_hbm.at[idx_s.at[k + j]], rows[bg], sem_g[bg]).wait()
        plsc.subcore_barrier()
        pltpu.sync_copy(acc_sh.at[pl.ds(sid * rz, rz)],
                        out_hbm.at[cid, pl.ds(sid * rz, rz)])

    return prop_kernel


def _xs_body(hist_ref, x_ref, o_ref, d_ref):
    deg = jnp.sum(hist_ref[...], axis=1) + 1.0
    dinv = lax.rsqrt(deg)
    xs = x_ref[...] * dinv[:, None]
    fh = xs.shape[1] // 2
    o_ref[0] = xs[:, :fh]
    o_ref[1] = xs[:, fh:]
    d_ref[...] = jnp.broadcast_to(dinv[:, None], d_ref.shape)


def _mid_body(d_ref, part_ref, xs_ref, w1_ref, b1_ref, w2_ref, o_ref):
    dinv = d_ref[...][:, 0]
    agg = jnp.concatenate([part_ref[0] + xs_ref[0], part_ref[1] + xs_ref[1]],
                          axis=1)
    p1 = dinv[:, None] * agg
    h = jnp.maximum(
        jnp.dot(p1, w1_ref[...], preferred_element_type=jnp.float32)
        + b1_ref[0:1, :], 0.0)
    g = jnp.dot(h, w2_ref[...], preferred_element_type=jnp.float32)
    o_ref[...] = g * dinv[:, None]


def _final_body(d_ref, part_ref, gs_ref, b2_ref, o_ref):
    dinv = d_ref[...][:, 0]
    o_ref[...] = dinv[:, None] * (part_ref[0] + part_ref[1] + gs_ref[...]) \
        + b2_ref[0:1, :]


def kernel(x, edge_index, W1, b1, W2, b2):
    n, d_in = x.shape
    d_hid = W1.shape[1]
    n_cls = W2.shape[1]
    fh = d_in // 2
    src = edge_index[0]
    dst = edge_index[1]

    n_pad = -(-(n + 1) // 128) * 128

    src32, k32 = _pad_chunks(src, NW, 0)
    dst32, _ = _pad_chunks(dst, NW, n)
    src16, k16 = _pad_chunks(src, NS, 0)
    dst16, _ = _pad_chunks(dst, NS, n)
    src16sc = jnp.stack([src16, src16 + n])

    hist = _make_hist_kernel(n_pad, k32)(dst32)
    hist_n = hist[:, :n].T

    br = 1000
    grid = (n // br,)
    hist_spec = pl.BlockSpec((br, NW), lambda i: (i, 0))
    row_spec = lambda f: pl.BlockSpec((br, f), lambda i: (i, 0))
    part_spec = lambda f: pl.BlockSpec((NC, br, f), lambda i: (0, i, 0))

    xs2, dinv8 = pl.pallas_call(
        _xs_body,
        grid=grid,
        in_specs=[hist_spec, row_spec(d_in)],
        out_specs=[part_spec(fh), row_spec(8)],
        out_shape=[jax.ShapeDtypeStruct((NC, n, fh), jnp.float32),
                   jax.ShapeDtypeStruct((n, 8), jnp.float32)],
    )(hist_n, x)

    zeros_h = jnp.zeros((n_pad, fh), jnp.float32)
    part1 = _make_prop_kernel(n_pad, k16, fh, True)(
        src16sc, dst16, xs2.reshape(NC * n, fh), zeros_h)

    b1r = jnp.broadcast_to(b1[None, :], (8, d_hid))
    full = lambda shape: pl.BlockSpec(shape, lambda i: tuple(0 for _ in shape))
    gs = pl.pallas_call(
        _mid_body,
        grid=grid,
        in_specs=[row_spec(8), part_spec(fh), part_spec(fh),
                  full((d_in, d_hid)), full((8, d_hid)), full((d_hid, n_cls))],
        out_specs=row_spec(n_cls),
        out_shape=jax.ShapeDtypeStruct((n, n_cls), jnp.float32),
    )(dinv8, part1, xs2, W1, b1r, W2)

    zeros_cls = jnp.zeros((n_pad, n_cls), jnp.float32)
    part2 = _make_prop_kernel(n_pad, k32, n_cls, False)(
        src32, dst32, gs, zeros_cls)

    b2r = jnp.broadcast_to(b2[None, :], (8, n_cls))
    out = pl.pallas_call(
        _final_body,
        grid=grid,
        in_specs=[row_spec(8), part_spec(n_cls), row_spec(n_cls),
                  full((8, n_cls))],
        out_specs=row_spec(n_cls),
        out_shape=jax.ShapeDtypeStruct((n, n_cls), jnp.float32),
    )(dinv8, part2, gs, b2r)
    return out


# speedup vs baseline: 1.4358x; 1.0117x over previous
import functools

import jax
import jax.numpy as jnp
from jax import lax
from jax.experimental import pallas as pl
from jax.experimental.pallas import tpu as pltpu, tpu_sc as plsc

NC = 2
NS = 16
NW = NC * NS
C = 128
NBUF = 4

_SC_PARAMS = pltpu.CompilerParams(
    needs_layout_passes=False, use_tc_tiling_on_sc=False)


def _pad_chunks(arr, workers, fill, nbuf=NBUF):
    e = arr.shape[0]
    ept = e // workers
    k = -(-(-(-ept // C)) // nbuf) * nbuf
    kt = k + nbuf
    pad = kt * C - ept
    out = jnp.concatenate(
        [arr.reshape(workers, ept),
         jnp.full((workers, pad), fill, jnp.int32)], axis=1)
    return out.reshape(workers, kt, C), k


def _make_hist_kernel(n_pad, k):
    mesh = plsc.VectorSubcoreMesh(core_axis_name="c", subcore_axis_name="s")

    @functools.partial(
        pl.kernel,
        out_type=jax.ShapeDtypeStruct((NW, n_pad), jnp.float32),
        mesh=mesh,
        scratch_types=[
            pltpu.VMEM((n_pad,), jnp.float32),
            pltpu.VMEM((k, C), jnp.int32),
        ],
        compiler_params=_SC_PARAMS,
    )
    def hist_kernel(dst_hbm, out_hbm, hist_v, idx_v):
        cid = lax.axis_index("c")
        sid = lax.axis_index("s")
        wid = sid * NC + cid

        pltpu.sync_copy(dst_hbm.at[wid, pl.ds(0, k)], idx_v)

        def zero_body(i, _):
            hist_v[pl.ds(i * 16, 16)] = jnp.zeros((16,), jnp.float32)
            return 0

        lax.fori_loop(0, n_pad // 16, zero_body, 0)

        ones = jnp.ones((16,), jnp.float32)

        def chunk_body(kk, _):
            for j in range(C // 16):
                iv = idx_v[kk, pl.ds(j * 16, 16)]
                plsc.addupdate_scatter(hist_v, [iv], ones)
            return 0

        lax.fori_loop(0, k, chunk_body, 0)
        pltpu.sync_copy(hist_v, out_hbm.at[wid])

    return hist_kernel


def _make_prop_kernel(n_pad, k, feat, feature_split, nbuf=NBUF):
    mesh = plsc.VectorSubcoreMesh(core_axis_name="c", subcore_axis_name="s")
    rz = n_pad // NS
    H = nbuf // 2
    NH = nbuf - H

    @functools.partial(
        pl.kernel,
        out_type=jax.ShapeDtypeStruct((NC, n_pad, feat), jnp.float32),
        mesh=mesh,
        scratch_types=[
            pltpu.VMEM((k + nbuf, C), jnp.int32),
            pltpu.VMEM((k + nbuf, C), jnp.int32),
            [pltpu.VMEM((C, feat), jnp.float32) for _ in range(nbuf)],
            [pltpu.SemaphoreType.DMA for _ in range(nbuf)],
            [pltpu.SemaphoreType.DMA for _ in range(nbuf)],
            pltpu.VMEM_SHARED((n_pad, feat), jnp.float32),
        ],
        compiler_params=_SC_PARAMS,
    )
    def prop_kernel(src_hbm, dst_hbm, table_hbm, zeros_hbm, out_hbm,
                    idx_s, idx_d, rows, sem_g, sem_s, acc_sh):
        cid = lax.axis_index("c")
        sid = lax.axis_index("s")

        if feature_split:
            pltpu.sync_copy(src_hbm.at[cid, sid], idx_s)
            pltpu.sync_copy(dst_hbm.at[sid], idx_d)
        else:
            wid = sid * NC + cid
            pltpu.sync_copy(src_hbm.at[wid], idx_s)
            pltpu.sync_copy(dst_hbm.at[wid], idx_d)
        for b in range(H):
            pltpu.async_copy(table_hbm.at[idx_s.at[b]], rows[b], sem_g[b])
        pltpu.sync_copy(zeros_hbm.at[pl.ds(sid * rz, rz)],
                        acc_sh.at[pl.ds(sid * rz, rz)])
        plsc.subcore_barrier()

        def stage(kk, b, skip_scatter_wait=False):
            bn = (b + H) % nbuf
            pltpu.make_async_copy(
                table_hbm.at[idx_s.at[kk]], rows[b], sem_g[b]).wait()
            pltpu.async_copy(rows[b], acc_sh.at[idx_d.at[kk]], sem_s[b],
                             add=True)
            if not skip_scatter_wait:
                pltpu.make_async_copy(
                    rows[bn], acc_sh.at[idx_d.at[0]], sem_s[bn]).wait()
            pltpu.async_copy(
                table_hbm.at[idx_s.at[kk + H]], rows[bn], sem_g[bn])

        for b in range(nbuf):
            stage(b, b, skip_scatter_wait=b < NH)

        def group_body(g, _):
            for b in range(nbuf):
                stage(g * nbuf + b, b)
            return 0

        lax.fori_loop(1, k // nbuf, group_body, 0)
        for j in range(NH):
            bs = (k - NH + j) % nbuf
            pltpu.make_async_copy(
                rows[bs], acc_sh.at[idx_d.at[0]], sem_s[bs]).wait()
        for j in range(H):
            bg = (k + j) % nbuf
            pltpu.make_async_copy(
                table_hbm.at[idx_s.at[k + j]], rows[bg], sem_g[bg]).wait()
        plsc.subcore_barrier()
        pltpu.sync_copy(acc_sh.at[pl.ds(sid * rz, rz)],
                        out_hbm.at[cid, pl.ds(sid * rz, rz)])

    return prop_kernel


def _xs_body(hist_ref, x_ref, o_ref, d_ref):
    deg = jnp.sum(hist_ref[...], axis=1) + 1.0
    dinv = lax.rsqrt(deg)
    xs = x_ref[...] * dinv[:, None]
    fh = xs.shape[1] // 2
    o_ref[0] = xs[:, :fh]
    o_ref[1] = xs[:, fh:]
    d_ref[...] = jnp.broadcast_to(dinv[:, None], d_ref.shape)


def _mid_body(d_ref, part_ref, xs_ref, w1_ref, b1_ref, w2_ref, o_ref):
    dinv = d_ref[...][:, 0]
    agg = jnp.concatenate([part_ref[0] + xs_ref[0], part_ref[1] + xs_ref[1]],
                          axis=1)
    p1 = dinv[:, None] * agg
    h = jnp.maximum(
        jnp.dot(p1, w1_ref[...], preferred_element_type=jnp.float32)
        + b1_ref[0:1, :], 0.0)
    g = jnp.dot(h, w2_ref[...], preferred_element_type=jnp.float32)
    o_ref[...] = g * dinv[:, None]


def _final_body(d_ref, part_ref, gs_ref, b2_ref, o_ref):
    dinv = d_ref[...][:, 0]
    o_ref[...] = dinv[:, None] * (part_ref[0] + part_ref[1] + gs_ref[...]) \
        + b2_ref[0:1, :]


def kernel(x, edge_index, W1, b1, W2, b2):
    n, d_in = x.shape
    d_hid = W1.shape[1]
    n_cls = W2.shape[1]
    fh = d_in // 2
    src = edge_index[0]
    dst = edge_index[1]

    n_pad = -(-(n + 1) // 128) * 128

    src32, k32 = _pad_chunks(src, NW, 0)
    dst32, _ = _pad_chunks(dst, NW, n)
    nb1 = 5
    src16, k16 = _pad_chunks(src, NS, 0, nb1)
    dst16, _ = _pad_chunks(dst, NS, n, nb1)
    src16sc = jnp.stack([src16, src16 + n])

    hist = _make_hist_kernel(n_pad, k32)(dst32)
    hist_n = hist[:, :n].T

    br = 2000
    grid = (n // br,)
    hist_spec = pl.BlockSpec((br, NW), lambda i: (i, 0))
    row_spec = lambda f: pl.BlockSpec((br, f), lambda i: (i, 0))
    part_spec = lambda f: pl.BlockSpec((NC, br, f), lambda i: (0, i, 0))

    xs2, dinv8 = pl.pallas_call(
        _xs_body,
        grid=grid,
        in_specs=[hist_spec, row_spec(d_in)],
        out_specs=[part_spec(fh), row_spec(8)],
        out_shape=[jax.ShapeDtypeStruct((NC, n, fh), jnp.float32),
                   jax.ShapeDtypeStruct((n, 8), jnp.float32)],
    )(hist_n, x)

    zeros_h = jnp.zeros((n_pad, fh), jnp.float32)
    part1 = _make_prop_kernel(n_pad, k16, fh, True, nb1)(
        src16sc, dst16, xs2.reshape(NC * n, fh), zeros_h)

    b1r = jnp.broadcast_to(b1[None, :], (8, d_hid))
    full = lambda shape: pl.BlockSpec(shape, lambda i: tuple(0 for _ in shape))
    gs = pl.pallas_call(
        _mid_body,
        grid=grid,
        in_specs=[row_spec(8), part_spec(fh), part_spec(fh),
                  full((d_in, d_hid)), full((8, d_hid)), full((d_hid, n_cls))],
        out_specs=row_spec(n_cls),
        out_shape=jax.ShapeDtypeStruct((n, n_cls), jnp.float32),
    )(dinv8, part1, xs2, W1, b1r, W2)

    zeros_cls = jnp.zeros((n_pad, n_cls), jnp.float32)
    part2 = _make_prop_kernel(n_pad, k32, n_cls, False)(
        src32, dst32, gs, zeros_cls)

    b2r = jnp.broadcast_to(b2[None, :], (8, n_cls))
    out = pl.pallas_call(
        _final_body,
        grid=grid,
        in_specs=[row_spec(8), part_spec(n_cls), row_spec(n_cls),
                  full((8, n_cls))],
        out_specs=row_spec(n_cls),
        out_shape=jax.ShapeDtypeStruct((n, n_cls), jnp.float32),
    )(dinv8, part2, gs, b2r)
    return out
